# baseline (speedup 1.0000x reference)
import jax
import jax.numpy as jnp
from jax.experimental import pallas as pl
from jax.experimental.pallas import tpu as pltpu

BATCH = 1024
FEAT = 64
BITS = 16
ENC = FEAT * BITS
HID = 2048
CLASSES = 10
SEG_THRESH = 4.0

CH = 512
NCH = HID // CH

_RHS1 = (((1,), (1,)), ((), ()))


def _fused(x_ref, w0_hbm, w1_hbm, w2_hbm, ow_ref, out_ref,
           w0_s, w1_s, w2_s, a0_s, a1_s, a2_s, sem):
    layers = ((w0_hbm, w0_s, 0), (w1_hbm, w1_s, 1), (w2_hbm, w2_s, 2))

    def _copy(gk):
        li, c = divmod(gk, NCH)
        hbm, ring, _ = layers[li]
        slot = c % 3
        return pltpu.make_async_copy(
            hbm.at[pl.ds(c * CH, CH), :],
            ring.at[slot],
            sem.at[li, slot],
        )

    _copy(0).start()
    _copy(1).start()

    k = jnp.floor(x_ref[:] * (BITS - 1.0))
    jf = jax.lax.broadcasted_iota(jnp.int32, (FEAT, ENC), 1)
    ff = jax.lax.broadcasted_iota(jnp.int32, (FEAT, ENC), 0)
    expand = (jf // BITS == ff).astype(jnp.float32)
    kr = jnp.dot(k, expand, preferred_element_type=jnp.float32)
    t = (jax.lax.broadcasted_iota(jnp.int32, (1, ENC), 1) % BITS
         ).astype(jnp.float32)
    a0_s[:] = (kr >= t).astype(jnp.float32)

    srcs = (a0_s, a1_s, a2_s)
    dsts = (a1_s, a2_s, None)
    total_copies = 3 * NCH
    scores = jnp.zeros((BATCH, CLASSES), dtype=jnp.float32)

    for gk in range(total_copies):
        li, c = divmod(gk, NCH)
        _, ring, _ = layers[li]
        slot = c % 3
        _copy(gk).wait()
        if gk + 2 < total_copies:
            _copy(gk + 2).start()
        z = jax.lax.dot_general(srcs[li][:], ring[slot], _RHS1,
                                preferred_element_type=jnp.float32)
        act = (z >= SEG_THRESH).astype(jnp.float32)
        if dsts[li] is not None:
            dsts[li][:, c * CH:(c + 1) * CH] = act
        scores = scores + jnp.dot(act, ow_ref[li, c * CH:(c + 1) * CH, :],
                                  preferred_element_type=jnp.float32)

    out_ref[:] = scores


def kernel(x, W0, W1, W2, outW):
    return pl.pallas_call(
        _fused,
        in_specs=[
            pl.BlockSpec((BATCH, FEAT), lambda: (0, 0)),
            pl.BlockSpec(memory_space=pl.ANY),
            pl.BlockSpec(memory_space=pl.ANY),
            pl.BlockSpec(memory_space=pl.ANY),
            pl.BlockSpec((3, HID, CLASSES), lambda: (0, 0, 0)),
        ],
        out_specs=pl.BlockSpec((BATCH, CLASSES), lambda: (0, 0)),
        out_shape=jax.ShapeDtypeStruct((BATCH, CLASSES), jnp.float32),
        scratch_shapes=[
            pltpu.VMEM((3, CH, ENC), jnp.float32),
            pltpu.VMEM((3, CH, HID), jnp.float32),
            pltpu.VMEM((3, CH, HID), jnp.float32),
            pltpu.VMEM((BATCH, ENC), jnp.float32),
            pltpu.VMEM((BATCH, HID), jnp.float32),
            pltpu.VMEM((BATCH, HID), jnp.float32),
            pltpu.SemaphoreType.DMA((3, 3)),
        ],
    )(x, W0, W1, W2, outW)
